# baseline (speedup 1.0000x reference)
import jax
import jax.numpy as jnp
from jax import lax
from jax.experimental import pallas as pl
from jax.experimental.pallas import tpu as pltpu
from jax.experimental.pallas import tpu_sc as plsc

B = 16
S = 4096
F = 128
Q = 512
BLK = 512
NW = 32
QW = (B * Q) // NW
L = 16


def _cumsum_body(tri_ref, x_ref, o_ref, carry_ref):
    j = pl.program_id(1)

    @pl.when(j == 0)
    def _():
        carry_ref[...] = jnp.zeros_like(carry_ref)

    x = x_ref[0]
    cs = jnp.dot(tri_ref[...], x, preferred_element_type=jnp.float32)
    cs = cs + carry_ref[...]
    o_ref[0] = cs
    carry_ref[...] = cs[BLK - 1:BLK, :]


def _cumsum_tc(x, tri):
    return pl.pallas_call(
        _cumsum_body,
        grid=(B, S // BLK),
        in_specs=[
            pl.BlockSpec((BLK, BLK), lambda b, j: (0, 0)),
            pl.BlockSpec((1, BLK, F), lambda b, j: (b, j, 0)),
        ],
        out_specs=pl.BlockSpec((1, BLK, F), lambda b, j: (b, j, 0)),
        out_shape=jax.ShapeDtypeStruct((B, S, F), jnp.float32),
        scratch_shapes=[pltpu.VMEM((1, F), jnp.float32)],
        compiler_params=pltpu.CompilerParams(
            dimension_semantics=("arbitrary", "arbitrary"),
        ),
    )(tri, x)


def _gather_mean_body(cs_hbm, mi_hbm, out_hbm,
                      sraw_v, eraw_v, sidx_v, eidx_v, rows_s, rows_e, rcp_v,
                      out_v, sem_s, sem_e):
    wid = lax.axis_index("s") * 2 + lax.axis_index("c")
    b = wid // 2
    h = wid % 2
    qbase = wid * QW
    soff = b * (2 * Q) + h * QW

    pltpu.sync_copy(mi_hbm.at[pl.ds(soff, QW)], sraw_v)
    pltpu.sync_copy(mi_hbm.at[pl.ds(soff + Q, QW)], eraw_v)

    base = b * S

    def _flatten(i, _):
        sl = pl.ds(i * L, L)
        sv = jnp.minimum(jnp.maximum(sraw_v[sl], 0), S - 1)
        ev = jnp.minimum(jnp.maximum(eraw_v[sl], 0), S - 1)
        sidx_v[sl] = sv + base
        eidx_v[sl] = ev + base
        rcp_v[sl] = 1.0 / (ev - sv + 1).astype(jnp.float32)
        return 0

    lax.fori_loop(0, QW // L, _flatten, 0, unroll=4)

    cp_s = pltpu.async_copy(cs_hbm.at[sidx_v], rows_s, sem_s)
    cp_e = pltpu.async_copy(cs_hbm.at[eidx_v], rows_e, sem_e)
    cp_s.wait()
    cp_e.wait()

    def _mean(i, _):
        rcpc = rcp_v[pl.ds(i * L, L)]
        for jq in range(L):
            q = i * L + jq
            r = rcpc[jq]
            for c in range(F // L):
                sl = pl.ds(c * L, L)
                out_v[q, sl] = (rows_e[q, sl] - rows_s[q, sl]) * r
        return 0

    lax.fori_loop(0, QW // L, _mean, 0)

    pltpu.sync_copy(out_v, out_hbm.at[pl.ds(qbase, QW)])


def _gather_mean_sc(cs_flat, mi_flat):
    mesh = plsc.VectorSubcoreMesh(core_axis_name="c", subcore_axis_name="s")
    return pl.kernel(
        _gather_mean_body,
        mesh=mesh,
        out_type=jax.ShapeDtypeStruct((B * Q, F), jnp.float32),
        scratch_types=[
            pltpu.VMEM((QW,), jnp.int32),
            pltpu.VMEM((QW,), jnp.int32),
            pltpu.VMEM((QW,), jnp.int32),
            pltpu.VMEM((QW,), jnp.int32),
            pltpu.VMEM((QW, F), jnp.float32),
            pltpu.VMEM((QW, F), jnp.float32),
            pltpu.VMEM((QW,), jnp.float32),
            pltpu.VMEM((QW, F), jnp.float32),
            pltpu.SemaphoreType.DMA,
            pltpu.SemaphoreType.DMA,
        ],
    )(cs_flat, mi_flat)


def kernel(opcode_filters, method_indices):
    tri = jnp.tril(jnp.ones((BLK, BLK), jnp.float32))
    mi_t = method_indices.transpose(0, 2, 1)
    cs = _cumsum_tc(opcode_filters, tri)
    out = _gather_mean_sc(cs.reshape(B * S, F), mi_t.reshape(B * 2 * Q))
    return out.reshape(B, Q, F)


# speedup vs baseline: 1.0111x; 1.0111x over previous
import jax
import jax.numpy as jnp
from jax import lax
from jax.experimental import pallas as pl
from jax.experimental.pallas import tpu as pltpu
from jax.experimental.pallas import tpu_sc as plsc

B = 16
S = 4096
F = 128
Q = 512
BLK = 512
NW = 32
QW = (B * Q) // NW
L = 16


def _cumsum_body(x_ref, o_ref, carry_ref, tri_ref):
    b = pl.program_id(0)
    j = pl.program_id(1)

    @pl.when(jnp.logical_and(b == 0, j == 0))
    def _():
        row = lax.broadcasted_iota(jnp.int32, (BLK, BLK), 0)
        col = lax.broadcasted_iota(jnp.int32, (BLK, BLK), 1)
        tri_ref[...] = jnp.where(row >= col, 1.0, 0.0).astype(jnp.float32)

    @pl.when(j == 0)
    def _():
        carry_ref[...] = jnp.zeros_like(carry_ref)

    x = x_ref[0]
    cs = jnp.dot(tri_ref[...], x, preferred_element_type=jnp.float32)
    cs = cs + carry_ref[...]
    o_ref[0] = cs
    carry_ref[...] = cs[BLK - 1:BLK, :]


def _cumsum_tc(x):
    return pl.pallas_call(
        _cumsum_body,
        grid=(B, S // BLK),
        in_specs=[
            pl.BlockSpec((1, BLK, F), lambda b, j: (b, j, 0)),
        ],
        out_specs=pl.BlockSpec((1, BLK, F), lambda b, j: (b, j, 0)),
        out_shape=jax.ShapeDtypeStruct((B, S, F), jnp.float32),
        scratch_shapes=[
            pltpu.VMEM((1, F), jnp.float32),
            pltpu.VMEM((BLK, BLK), jnp.float32),
        ],
        compiler_params=pltpu.CompilerParams(
            dimension_semantics=("arbitrary", "arbitrary"),
        ),
    )(x)


def _gather_mean_body(cs_hbm, mi_hbm, out_hbm,
                      sraw_v, eraw_v, sidx_v, eidx_v, rows_s, rows_e, rcp_v,
                      out_v, sem_s, sem_e):
    wid = lax.axis_index("s") * 2 + lax.axis_index("c")
    b = wid // 2
    h = wid % 2
    qbase = wid * QW
    soff = b * (2 * Q) + h * QW

    pltpu.sync_copy(mi_hbm.at[pl.ds(soff, QW)], sraw_v)
    pltpu.sync_copy(mi_hbm.at[pl.ds(soff + Q, QW)], eraw_v)

    base = b * S

    def _flatten(i, _):
        sl = pl.ds(i * L, L)
        sv = jnp.minimum(jnp.maximum(sraw_v[sl], 0), S - 1)
        ev = jnp.minimum(jnp.maximum(eraw_v[sl], 0), S - 1)
        sidx_v[sl] = sv + base
        eidx_v[sl] = ev + base
        rcp_v[sl] = 1.0 / (ev - sv + 1).astype(jnp.float32)
        return 0

    lax.fori_loop(0, QW // L, _flatten, 0, unroll=4)

    cp_s = pltpu.async_copy(cs_hbm.at[sidx_v], rows_s, sem_s)
    cp_e = pltpu.async_copy(cs_hbm.at[eidx_v], rows_e, sem_e)
    cp_s.wait()
    cp_e.wait()

    def _mean(i, _):
        rcpc = rcp_v[pl.ds(i * L, L)]
        for jq in range(L):
            q = i * L + jq
            r = rcpc[jq]
            for c in range(F // L):
                sl = pl.ds(c * L, L)
                out_v[q, sl] = (rows_e[q, sl] - rows_s[q, sl]) * r
        return 0

    lax.fori_loop(0, QW // L, _mean, 0)

    pltpu.sync_copy(out_v, out_hbm.at[pl.ds(qbase, QW)])


def _gather_mean_sc(cs_flat, mi_flat):
    mesh = plsc.VectorSubcoreMesh(core_axis_name="c", subcore_axis_name="s")
    return pl.kernel(
        _gather_mean_body,
        mesh=mesh,
        out_type=jax.ShapeDtypeStruct((B * Q, F), jnp.float32),
        scratch_types=[
            pltpu.VMEM((QW,), jnp.int32),
            pltpu.VMEM((QW,), jnp.int32),
            pltpu.VMEM((QW,), jnp.int32),
            pltpu.VMEM((QW,), jnp.int32),
            pltpu.VMEM((QW, F), jnp.float32),
            pltpu.VMEM((QW, F), jnp.float32),
            pltpu.VMEM((QW,), jnp.float32),
            pltpu.VMEM((QW, F), jnp.float32),
            pltpu.SemaphoreType.DMA,
            pltpu.SemaphoreType.DMA,
        ],
    )(cs_flat, mi_flat)


def kernel(opcode_filters, method_indices):
    mi_t = method_indices.transpose(0, 2, 1)
    cs = _cumsum_tc(opcode_filters)
    out = _gather_mean_sc(cs.reshape(B * S, F), mi_t.reshape(B * 2 * Q))
    return out.reshape(B, Q, F)


# speedup vs baseline: 1.6351x; 1.6171x over previous
import jax
import jax.numpy as jnp
from jax import lax
from jax.experimental import pallas as pl
from jax.experimental.pallas import tpu as pltpu
from jax.experimental.pallas import tpu_sc as plsc

B = 16
S = 4096
F = 128
Q = 512
L = 16
CPW = 4


def _sc_body(x_hbm, mi_hbm, out_hbm, x_v, sraw_v, eraw_v, rcp_v, out_v):
    wid = lax.axis_index("s") * 2 + lax.axis_index("c")
    b = wid // 2
    half = wid % 2

    pltpu.sync_copy(mi_hbm.at[pl.ds(b * (2 * Q), Q)], sraw_v)
    pltpu.sync_copy(mi_hbm.at[pl.ds(b * (2 * Q) + Q, Q)], eraw_v)

    def _prep(i, _):
        sl = pl.ds(i * L, L)
        sv = jnp.minimum(jnp.maximum(sraw_v[sl], 0), S - 1)
        ev = jnp.minimum(jnp.maximum(eraw_v[sl], 0), S - 1)
        sraw_v[sl] = sv
        eraw_v[sl] = ev
        rcp_v[sl] = 1.0 / (ev - sv + 1).astype(jnp.float32)
        return 0

    lax.fori_loop(0, Q // L, _prep, 0, unroll=4)

    for k in range(CPW):
        c = half * CPW + k

        pltpu.sync_copy(x_hbm.at[b, :, pl.ds(c * L, L)], x_v)

        def _cs(r, acc):
            acc = acc + x_v[r, :]
            x_v[r, :] = acc
            return acc

        lax.fori_loop(0, S, _cs, jnp.zeros((L,), jnp.float32), unroll=8)

        def _g(i, _):
            sv = sraw_v[pl.ds(i * L, L)]
            ev = eraw_v[pl.ds(i * L, L)]
            rv = rcp_v[pl.ds(i * L, L)]
            for jq in range(L):
                q = i * L + jq
                out_v[q, :] = (x_v[ev[jq], :] - x_v[sv[jq], :]) * rv[jq]
            return 0

        lax.fori_loop(0, Q // L, _g, 0)

        pltpu.sync_copy(out_v, out_hbm.at[b, :, pl.ds(c * L, L)])


def _method_features_sc(x, mi_flat):
    mesh = plsc.VectorSubcoreMesh(core_axis_name="c", subcore_axis_name="s")
    return pl.kernel(
        _sc_body,
        mesh=mesh,
        out_type=jax.ShapeDtypeStruct((B, Q, F), jnp.float32),
        compiler_params=pltpu.CompilerParams(use_tc_tiling_on_sc=False),
        scratch_types=[
            pltpu.VMEM((S, L), jnp.float32),
            pltpu.VMEM((Q,), jnp.int32),
            pltpu.VMEM((Q,), jnp.int32),
            pltpu.VMEM((Q,), jnp.float32),
            pltpu.VMEM((Q, L), jnp.float32),
        ],
    )(x, mi_flat)


def kernel(opcode_filters, method_indices):
    mi_t = method_indices.transpose(0, 2, 1)
    return _method_features_sc(opcode_filters, mi_t.reshape(B * 2 * Q))
